# baseline (speedup 1.0000x reference)
import jax
import jax.numpy as jnp
from jax.experimental import pallas as pl
from jax.experimental.pallas import tpu as pltpu

D_FEAT = 6
HID = 64
SEQ = 60
N = 1024
DAYS = 500
STOCKS = 1024
KTOP = 10
ROW_BLK = 128

_F32 = jnp.float32


def _mm_t(a, b):
    return jax.lax.dot_general(a, b, (((1,), (1,)), ((), ())),
                               preferred_element_type=_F32)


def _mm(a, b):
    return jax.lax.dot_general(a, b, (((1,), (0,)), ((), ())),
                               preferred_element_type=_F32)


def _mm_tl(a, b):
    return jax.lax.dot_general(a, b, (((0,), (0,)), ((), ())),
                               preferred_element_type=_F32)


def _leaky(v):
    return jnp.where(v >= 0, v, 0.01 * v)


def _encoder_kernel(xb3_ref, W0rzi_ref, W0ni_ref, Whh0rz_ref, Whh0n_ref,
                    W1rz_ref, W1ni_ref, W1nh_ref,
                    brz0_ref, bin0_ref, bhn0_ref,
                    brz1_ref, bin1_ref, bhn1_ref,
                    lin0W_ref, lin0b_ref, lin1W_ref, lin1b_ref,
                    proj1W_ref, thd_ref,
                    mboT_ref, p1T_ref, dayidx_ref):
    W0rzi = W0rzi_ref[...]
    W0ni = W0ni_ref[...]
    Whh0rz = Whh0rz_ref[...]
    Whh0n = Whh0n_ref[...]
    W1rz = W1rz_ref[...]
    W1ni = W1ni_ref[...]
    W1nh = W1nh_ref[...]
    brz0 = brz0_ref[...]
    bin0 = bin0_ref[...]
    bhn0 = bhn0_ref[...]
    brz1 = brz1_ref[...]
    bin1 = bin1_ref[...]
    bhn1 = bhn1_ref[...]

    def body(t, carry):
        h0, h1 = carry
        x_t = xb3_ref[pl.ds(t, 1), :, :].reshape(D_FEAT, N)
        grz = _mm(W0rzi, x_t) + brz0
        gn = _mm(W0ni, x_t) + bin0
        a0 = jax.nn.sigmoid(grz + _mm(Whh0rz, h0))
        r0 = a0[:HID]
        z0 = a0[HID:]
        n0 = jnp.tanh(gn + r0 * (_mm(Whh0n, h0) + bhn0))
        h0 = (1.0 - z0) * n0 + z0 * h0
        cat = jnp.concatenate([h0, h1], axis=0)
        a1 = jax.nn.sigmoid(_mm(W1rz, cat) + brz1)
        r1 = a1[:HID]
        z1 = a1[HID:]
        n1 = jnp.tanh(_mm(W1ni, h0) + bin1
                      + r1 * (_mm(W1nh, h1) + bhn1))
        h1 = (1.0 - z1) * n1 + z1 * h1
        return (h0, h1)

    h0 = jnp.zeros((HID, N), _F32)
    h1 = jnp.zeros((HID, N), _F32)
    _, out = jax.lax.fori_loop(0, SEQ, body, (h0, h1))

    mboT = _leaky(_mm(lin0W_ref[...], out) + lin0b_ref[...])
    mboT = _leaky(_mm(lin1W_ref[...], mboT) + lin1b_ref[...])
    mboT_ref[...] = mboT
    p1T_ref[...] = _mm(proj1W_ref[...], mboT)

    thd = thd_ref[...]
    mbd = _mm_t(jnp.full((1, N), 1.0 / N, _F32), mboT)
    num = _mm_t(mbd, thd)
    an = jnp.sqrt(jnp.sum(mbd * mbd))
    bn = jnp.sqrt(_mm_t(jnp.ones((1, HID), _F32), thd * thd))
    sim = num / (an * bn + 1e-6)

    iota = jax.lax.broadcasted_iota(jnp.int32, (1, DAYS), 1)
    lane = jax.lax.broadcasted_iota(jnp.int32, (1, 128), 1)
    vec = jnp.zeros((1, 128), jnp.int32)
    work = sim
    for i in range(KTOP):
        m = jnp.max(work)
        idx = jnp.min(jnp.where(work == m, iota, jnp.int32(2 ** 30)))
        vec = jnp.where(lane == i, idx, vec)
        work = jnp.where(iota == idx, -1e30, work)
    dayidx_ref[...] = vec


def _gather_proj_kernel(dayidx_ref, th_ref, lin0W_ref, lin0b_ref,
                        lin1W_ref, lin1b_ref, proj2W_ref, p2_ref):
    del dayidx_ref
    s = th_ref[0]
    s = _leaky(_mm_t(s, lin0W_ref[...]) + lin0b_ref[...])
    s = _leaky(_mm_t(s, lin1W_ref[...]) + lin1b_ref[...])
    p2_ref[0] = _mm_t(s, proj2W_ref[...])


def _knn_kernel(p1T_ref, mboT_ref, p2_ref, fcWa_ref, fcWb_ref, fcb_ref,
                pred_ref):
    p1tb = p1T_ref[...]
    p2 = p2_ref[...]
    numT = _mm(p2, p1tb)
    n1 = jnp.sqrt(_mm(jnp.ones((1, HID), _F32), p1tb * p1tb))
    n2 = jnp.sqrt(jnp.sum(p2 * p2, axis=1, keepdims=True))
    csT = numT / (n2 * n1 + 1e-6)

    work = csT
    thresh = None
    for i in range(KTOP):
        thresh = jnp.max(work, axis=0, keepdims=True)
        if i < KTOP - 1:
            work = jnp.where(work == thresh, -1e30, work)
    w = jnp.where(csT >= thresh, csT, 0.0)
    aggT = _mm_tl(p2, w)
    pred = (_mm(fcWa_ref[...], mboT_ref[...])
            + _mm(fcWb_ref[...], aggT) + fcb_ref[...])
    pred_ref[...] = pred.reshape(1, 1, ROW_BLK)


def kernel(x, train_hidden, train_hidden_day, W_ih0, W_hh0, b_ih0, b_hh0,
           W_ih1, W_hh1, b_ih1, b_hh1, lin0_W, lin0_b, lin1_W, lin1_b,
           proj1_W, proj2_W, fc_W, fc_b, k_day, n_neighbor):
    del k_day
    xb3 = x.reshape(N, D_FEAT, SEQ).transpose(2, 1, 0)

    W0rzi = W_ih0[:2 * HID]
    W0ni = W_ih0[2 * HID:]
    Whh0rz = W_hh0[:2 * HID]
    Whh0n = W_hh0[2 * HID:]
    W1rz = jnp.concatenate([W_ih1[:2 * HID], W_hh1[:2 * HID]], axis=1)
    W1ni = W_ih1[2 * HID:]
    W1nh = W_hh1[2 * HID:]
    brz0 = (b_ih0[:2 * HID] + b_hh0[:2 * HID]).reshape(2 * HID, 1)
    bin0 = b_ih0[2 * HID:].reshape(HID, 1)
    bhn0 = b_hh0[2 * HID:].reshape(HID, 1)
    brz1 = (b_ih1[:2 * HID] + b_hh1[:2 * HID]).reshape(2 * HID, 1)
    bin1 = b_ih1[2 * HID:].reshape(HID, 1)
    bhn1 = b_hh1[2 * HID:].reshape(HID, 1)

    mboT, p1T, dayvec = pl.pallas_call(
        _encoder_kernel,
        out_shape=[
            jax.ShapeDtypeStruct((HID, N), _F32),
            jax.ShapeDtypeStruct((HID, N), _F32),
            jax.ShapeDtypeStruct((1, 128), jnp.int32),
        ],
    )(xb3, W0rzi, W0ni, Whh0rz, Whh0n, W1rz, W1ni, W1nh,
      brz0, bin0, bhn0, brz1, bin1, bhn1,
      lin0_W, lin0_b.reshape(HID, 1), lin1_W, lin1_b.reshape(HID, 1),
      proj1_W, train_hidden_day)

    day_idx = dayvec[0, :KTOP]

    grid_spec = pltpu.PrefetchScalarGridSpec(
        num_scalar_prefetch=1,
        grid=(KTOP,),
        in_specs=[
            pl.BlockSpec((1, STOCKS, HID), lambda i, idx: (idx[i], 0, 0)),
            pl.BlockSpec((HID, HID), lambda i, idx: (0, 0)),
            pl.BlockSpec((HID,), lambda i, idx: (0,)),
            pl.BlockSpec((HID, HID), lambda i, idx: (0, 0)),
            pl.BlockSpec((HID,), lambda i, idx: (0,)),
            pl.BlockSpec((HID, HID), lambda i, idx: (0, 0)),
        ],
        out_specs=pl.BlockSpec((1, STOCKS, HID), lambda i, idx: (i, 0, 0)),
    )
    p2 = pl.pallas_call(
        _gather_proj_kernel,
        grid_spec=grid_spec,
        out_shape=jax.ShapeDtypeStruct((KTOP, STOCKS, HID), _F32),
    )(day_idx, train_hidden, lin0_W, lin0_b, lin1_W, lin1_b, proj2_W)

    p2f = p2.reshape(KTOP * STOCKS, HID)
    fcWa = fc_W[:, :HID]
    fcWb = fc_W[:, HID:] / n_neighbor
    fcb = fc_b.reshape(1, 1)

    n_blk = N // ROW_BLK
    pred = pl.pallas_call(
        _knn_kernel,
        grid=(n_blk,),
        in_specs=[
            pl.BlockSpec((HID, ROW_BLK), lambda i: (0, i)),
            pl.BlockSpec((HID, ROW_BLK), lambda i: (0, i)),
            pl.BlockSpec((KTOP * STOCKS, HID), lambda i: (0, 0)),
            pl.BlockSpec((1, HID), lambda i: (0, 0)),
            pl.BlockSpec((1, HID), lambda i: (0, 0)),
            pl.BlockSpec((1, 1), lambda i: (0, 0)),
        ],
        out_specs=pl.BlockSpec((1, 1, ROW_BLK), lambda i: (i, 0, 0)),
        out_shape=jax.ShapeDtypeStruct((n_blk, 1, ROW_BLK), _F32),
    )(p1T, mboT, p2f, fcWa, fcWb, fcb)

    return pred.reshape(N)


# speedup vs baseline: 1.1923x; 1.1923x over previous
import jax
import jax.numpy as jnp
from jax.experimental import pallas as pl
from jax.experimental.pallas import tpu as pltpu

D_FEAT = 6
HID = 64
SEQ = 60
N = 1024
DAYS = 500
STOCKS = 1024
KTOP = 10
ROW_BLK = 128

_F32 = jnp.float32


def _mm_t(a, b):
    return jax.lax.dot_general(a, b, (((1,), (1,)), ((), ())),
                               preferred_element_type=_F32)


def _mm(a, b):
    return jax.lax.dot_general(a, b, (((1,), (0,)), ((), ())),
                               preferred_element_type=_F32)


def _mm_tl(a, b):
    return jax.lax.dot_general(a, b, (((0,), (0,)), ((), ())),
                               preferred_element_type=_F32)


def _leaky(v):
    return jnp.where(v >= 0, v, 0.01 * v)


def _encoder_kernel(xT_ref, W0rzi_ref, W0ni_ref, Whh0rz_ref, Whh0n_ref,
                    W1rzi_ref, W1rzh_ref, W1ni_ref, W1nh_ref,
                    birz0_ref, bhrz0_ref, bin0_ref, bhn0_ref,
                    birz1_ref, bhrz1_ref, bin1_ref, bhn1_ref,
                    lin0W_ref, lin0b_ref, lin1W_ref, lin1b_ref,
                    proj1W_ref, thd_ref,
                    mbo_ref, p1_ref, dayidx_ref):
    W0rzi = W0rzi_ref[...]
    W0ni = W0ni_ref[...]
    Whh0rz = Whh0rz_ref[...]
    Whh0n = Whh0n_ref[...]
    W1rzi = W1rzi_ref[...]
    W1rzh = W1rzh_ref[...]
    W1ni = W1ni_ref[...]
    W1nh = W1nh_ref[...]
    birz0 = birz0_ref[...]
    bhrz0 = bhrz0_ref[...]
    bin0 = bin0_ref[...]
    bhn0 = bhn0_ref[...]
    birz1 = birz1_ref[...]
    bhrz1 = bhrz1_ref[...]
    bin1 = bin1_ref[...]
    bhn1 = bhn1_ref[...]

    def cell0(x_t, h0):
        a0 = jax.nn.sigmoid((_mm(W0rzi, x_t) + birz0)
                            + (_mm(Whh0rz, h0) + bhrz0))
        r0 = a0[:HID]
        z0 = a0[HID:]
        n0 = jnp.tanh((_mm(W0ni, x_t) + bin0)
                      + r0 * (_mm(Whh0n, h0) + bhn0))
        return (1.0 - z0) * n0 + z0 * h0

    def cell1(h0, h1):
        a1 = jax.nn.sigmoid((_mm(W1rzi, h0) + birz1)
                            + (_mm(W1rzh, h1) + bhrz1))
        r1 = a1[:HID]
        z1 = a1[HID:]
        n1 = jnp.tanh((_mm(W1ni, h0) + bin1)
                      + r1 * (_mm(W1nh, h1) + bhn1))
        return (1.0 - z1) * n1 + z1 * h1

    NH = N // 2

    def body(t, carry):
        h0a, h0b, h1a, h1b = carry
        x_t = jnp.concatenate(
            [xT_ref[pl.ds(t + SEQ * d, 1), :] for d in range(D_FEAT)],
            axis=0)
        xa = x_t[:, :NH]
        xb = x_t[:, NH:]
        nh0a = cell0(xa, h0a)
        nh0b = cell0(xb, h0b)
        c1a = cell1(h0a, h1a)
        c1b = cell1(h0b, h1b)
        first = t == 0
        nh1a = jnp.where(first, h1a, c1a)
        nh1b = jnp.where(first, h1b, c1b)
        return (nh0a, nh0b, nh1a, nh1b)

    zed = jnp.zeros((HID, NH), _F32)
    h0a, h0b, h1a, h1b = jax.lax.fori_loop(
        0, SEQ, body, (zed, zed, zed, zed))
    out = jnp.concatenate([cell1(h0a, h1a), cell1(h0b, h1b)], axis=1)

    mboT = _leaky(_mm(lin0W_ref[...], out) + lin0b_ref[...])
    mboT = _leaky(_mm(lin1W_ref[...], mboT) + lin1b_ref[...])
    p1T = _mm(proj1W_ref[...], mboT)
    iden = jnp.eye(HID, dtype=_F32)
    mbo_ref[...] = _mm_tl(mboT, iden)
    p1_ref[...] = _mm_tl(p1T, iden)

    thd = thd_ref[...]
    mbd = _mm_t(jnp.ones((1, N), _F32), mboT) * _F32(1.0 / N)
    num = _mm_t(thd, mbd)
    an = jnp.sqrt(jnp.sum(mbd * mbd))
    bn = jnp.sqrt(jnp.sum(thd * thd, axis=1, keepdims=True))
    sim = num / (an * bn + 1e-6)

    iota = jax.lax.broadcasted_iota(jnp.int32, (DAYS, 1), 0)
    lane = jax.lax.broadcasted_iota(jnp.int32, (1, 128), 1)
    vec = jnp.zeros((1, 128), jnp.int32)
    work = sim
    for i in range(KTOP):
        m = jnp.max(work)
        idx = jnp.min(jnp.where(work == m, iota, jnp.int32(2 ** 30)))
        vec = jnp.where(lane == i, idx, vec)
        work = jnp.where(iota == idx, -1e30, work)
    dayidx_ref[...] = vec


def _gather_proj_kernel(dayidx_ref, th_ref, lin0W_ref, lin0b_ref,
                        lin1W_ref, lin1b_ref, proj2W_ref, p2_ref, n2_ref):
    del dayidx_ref
    s = th_ref[0]
    s = _leaky(_mm_t(s, lin0W_ref[...]) + lin0b_ref[...])
    s = _leaky(_mm_t(s, lin1W_ref[...]) + lin1b_ref[...])
    p2 = _mm_t(s, proj2W_ref[...])
    p2_ref[0] = p2
    n2_ref[0] = jnp.sqrt(jnp.sum(p2 * p2, axis=1, keepdims=True))


def _knn_kernel(p1_ref, mbo_ref, p2_ref, n2_ref, fcWa_ref, fcWb_ref,
                fcb_ref, pred_ref):
    p1b = p1_ref[...]
    p2 = p2_ref[...]
    num = _mm_t(p1b, p2)
    n1 = jnp.sqrt(jnp.sum(p1b * p1b, axis=1, keepdims=True))
    n2 = n2_ref[...]
    cs = num / (n1 * n2 + 1e-6)

    work = cs
    thresh = None
    for i in range(KTOP):
        thresh = jnp.max(work, axis=1, keepdims=True)
        if i < KTOP - 1:
            work = jnp.where(work == thresh, -1e30, work)
    w = jnp.where(cs >= thresh, cs, 0.0)
    agg = _mm(w, p2)
    pred = (_mm_t(fcWa_ref[...], mbo_ref[...])
            + _mm_t(fcWb_ref[...], agg) + fcb_ref[...])
    pred_ref[...] = pred.reshape(1, 1, ROW_BLK)


def kernel(x, train_hidden, train_hidden_day, W_ih0, W_hh0, b_ih0, b_hh0,
           W_ih1, W_hh1, b_ih1, b_hh1, lin0_W, lin0_b, lin1_W, lin1_b,
           proj1_W, proj2_W, fc_W, fc_b, k_day, n_neighbor):
    del k_day

    W0rzi = W_ih0[:2 * HID]
    W0ni = W_ih0[2 * HID:]
    Whh0rz = W_hh0[:2 * HID]
    Whh0n = W_hh0[2 * HID:]
    W1rzi = W_ih1[:2 * HID]
    W1rzh = W_hh1[:2 * HID]
    W1ni = W_ih1[2 * HID:]
    W1nh = W_hh1[2 * HID:]
    birz0 = b_ih0[:2 * HID].reshape(2 * HID, 1)
    bhrz0 = b_hh0[:2 * HID].reshape(2 * HID, 1)
    bin0 = b_ih0[2 * HID:].reshape(HID, 1)
    bhn0 = b_hh0[2 * HID:].reshape(HID, 1)
    birz1 = b_ih1[:2 * HID].reshape(2 * HID, 1)
    bhrz1 = b_hh1[:2 * HID].reshape(2 * HID, 1)
    bin1 = b_ih1[2 * HID:].reshape(HID, 1)
    bhn1 = b_hh1[2 * HID:].reshape(HID, 1)

    mbo, p1, dayvec = pl.pallas_call(
        _encoder_kernel,
        out_shape=[
            jax.ShapeDtypeStruct((N, HID), _F32),
            jax.ShapeDtypeStruct((N, HID), _F32),
            jax.ShapeDtypeStruct((1, 128), jnp.int32),
        ],
    )(jnp.transpose(x), W0rzi, W0ni, Whh0rz, Whh0n,
      W1rzi, W1rzh, W1ni, W1nh,
      birz0, bhrz0, bin0, bhn0, birz1, bhrz1, bin1, bhn1,
      lin0_W, lin0_b.reshape(HID, 1), lin1_W, lin1_b.reshape(HID, 1),
      proj1_W, train_hidden_day)

    day_idx = dayvec[0, :KTOP]

    grid_spec = pltpu.PrefetchScalarGridSpec(
        num_scalar_prefetch=1,
        grid=(KTOP,),
        in_specs=[
            pl.BlockSpec((1, STOCKS, HID), lambda i, idx: (idx[i], 0, 0)),
            pl.BlockSpec((HID, HID), lambda i, idx: (0, 0)),
            pl.BlockSpec((HID,), lambda i, idx: (0,)),
            pl.BlockSpec((HID, HID), lambda i, idx: (0, 0)),
            pl.BlockSpec((HID,), lambda i, idx: (0,)),
            pl.BlockSpec((HID, HID), lambda i, idx: (0, 0)),
        ],
        out_specs=[
            pl.BlockSpec((1, STOCKS, HID), lambda i, idx: (i, 0, 0)),
            pl.BlockSpec((1, STOCKS, 1), lambda i, idx: (i, 0, 0)),
        ],
    )
    p2, n2 = pl.pallas_call(
        _gather_proj_kernel,
        grid_spec=grid_spec,
        out_shape=[
            jax.ShapeDtypeStruct((KTOP, STOCKS, HID), _F32),
            jax.ShapeDtypeStruct((KTOP, STOCKS, 1), _F32),
        ],
    )(day_idx, train_hidden, lin0_W, lin0_b, lin1_W, lin1_b, proj2_W)

    p2f = p2.reshape(KTOP * STOCKS, HID)
    n2row = n2.reshape(1, KTOP * STOCKS)
    fcWa = fc_W[:, :HID]
    fcWb = fc_W[:, HID:] / n_neighbor
    fcb = fc_b.reshape(1, 1)

    n_blk = N // ROW_BLK
    pred = pl.pallas_call(
        _knn_kernel,
        grid=(n_blk,),
        in_specs=[
            pl.BlockSpec((ROW_BLK, HID), lambda i: (i, 0)),
            pl.BlockSpec((ROW_BLK, HID), lambda i: (i, 0)),
            pl.BlockSpec((KTOP * STOCKS, HID), lambda i: (0, 0)),
            pl.BlockSpec((1, KTOP * STOCKS), lambda i: (0, 0)),
            pl.BlockSpec((1, HID), lambda i: (0, 0)),
            pl.BlockSpec((1, HID), lambda i: (0, 0)),
            pl.BlockSpec((1, 1), lambda i: (0, 0)),
        ],
        out_specs=pl.BlockSpec((1, 1, ROW_BLK), lambda i: (i, 0, 0)),
        out_shape=jax.ShapeDtypeStruct((n_blk, 1, ROW_BLK), _F32),
    )(p1, mbo, p2f, n2row, fcWa, fcWb, fcb)

    return pred.reshape(N)
